# baseline (speedup 1.0000x reference)
import functools

import jax
import jax.numpy as jnp
from jax import lax
from jax.experimental import pallas as pl
from jax.experimental.pallas import tpu as pltpu
from jax.experimental.pallas import tpu_sc as plsc

N_NODES = 10000
N_EDGES = 320000
D_FEAT = 128
D_EDGE = 16

NC, NS = 2, 16
NW = NC * NS
CHUNK = 128
N_CHUNKS = N_EDGES // CHUNK
SCAT_CPS = 80
SCAT_PAD_CHUNKS = NW * SCAT_CPS

G_CPW = -(-N_CHUNKS // NW)
G_CPW_PAD = G_CPW + (G_CPW % 2)
IDX_SPAN = G_CPW_PAD * CHUNK
IDX_PAD = (NW - 1) * G_CPW * CHUNK + IDX_SPAN
IDX_RING = 16
ROW_CHUNK = 40
N_ROW_CHUNKS = N_NODES // ROW_CHUNK
ROW_CPS = -(-N_ROW_CHUNKS // NS)

_VMESH = plsc.VectorSubcoreMesh(
    core_axis_name="c", subcore_axis_name="s", num_cores=NC, num_subcores=NS)


def _premix_body(x_ref, w_ref, s_ref, d_ref):
    p = jnp.dot(x_ref[...].astype(jnp.bfloat16), w_ref[0],
                preferred_element_type=jnp.float32)
    s_ref[...] = p[:, :D_FEAT]
    d_ref[...] = p[:, D_FEAT:]


def _premix(x_cat, w_stack):
    n_tile = 1000
    grid = (2 * N_NODES // n_tile,)
    node_spec = pl.BlockSpec((n_tile, D_FEAT), lambda t: (t, 0))
    w_spec = pl.BlockSpec((1, D_FEAT, 2 * D_FEAT), lambda t: (t // 10, 0, 0))
    out = jax.ShapeDtypeStruct((2 * N_NODES, D_FEAT), jnp.float32)
    return pl.pallas_call(
        _premix_body,
        grid=grid,
        in_specs=[node_spec, w_spec],
        out_specs=[node_spec] * 2,
        out_shape=[out] * 2,
    )(x_cat, w_stack)


@functools.partial(
    pl.kernel,
    out_type=jax.ShapeDtypeStruct((N_EDGES, D_FEAT), jnp.float32),
    mesh=_VMESH,
    scratch_types=[
        pltpu.VMEM((IDX_SPAN,), jnp.int32),
        pltpu.VMEM((IDX_SPAN,), jnp.int32),
        [pltpu.VMEM((CHUNK, D_FEAT), jnp.float32)] * 2,
        [pltpu.VMEM((CHUNK, D_FEAT), jnp.float32)] * 2,
        [pltpu.SemaphoreType.DMA] * 2,
        [pltpu.SemaphoreType.DMA] * 2,
    ],
)
def _sc_gather(s_tab, d_tab, isrc_hbm, idst_hbm, out_hbm,
               ibuf_s, ibuf_d, sbufs, dbufs, sems_g, sems_w):
    wid = lax.axis_index("s") * NC + lax.axis_index("c")
    span0 = pl.multiple_of(wid * (G_CPW * CHUNK), 8)

    pltpu.sync_copy(isrc_hbm.at[pl.ds(span0, IDX_SPAN)], ibuf_s)
    pltpu.sync_copy(idst_hbm.at[pl.ds(span0, IDX_SPAN)], ibuf_d)

    def rvalid(j):
        return jnp.logical_and(j < G_CPW, wid * G_CPW + j < N_CHUNKS)

    def issue_gather(j, b):
        isl = pl.ds(pl.multiple_of(j * CHUNK, 8), CHUNK)
        pltpu.async_copy(s_tab.at[ibuf_s.at[isl]], sbufs[b], sems_g[b])
        pltpu.async_copy(d_tab.at[ibuf_d.at[isl]], dbufs[b], sems_g[b])

    def drain_gather(j, b):
        isl = pl.ds(pl.multiple_of(j * CHUNK, 8), CHUNK)
        pltpu.make_async_copy(s_tab.at[ibuf_s.at[isl]], sbufs[b],
                              sems_g[b]).wait()
        pltpu.make_async_copy(d_tab.at[ibuf_d.at[isl]], dbufs[b],
                              sems_g[b]).wait()

    def out_slice(j):
        base = pl.multiple_of((wid * G_CPW + j) * CHUNK, 8)
        return out_hbm.at[pl.ds(base, CHUNK)]

    issue_gather(0, 0)

    def step(j, b):
        @pl.when(rvalid(j))
        def _():
            drain_gather(j, b)

            @pl.when(jnp.logical_and(j >= 1, rvalid(j - 1)))
            def _():
                pltpu.make_async_copy(dbufs[1 - b], out_slice(j - 1),
                                      sems_w[1 - b]).wait()

            @pl.when(rvalid(j + 1))
            def _():
                issue_gather(j + 1, 1 - b)

            @pl.loop(0, CHUNK)
            def _(r):
                for cc in range(0, D_FEAT, 16):
                    sl = pl.ds(cc, 16)
                    dbufs[b][r, sl] = sbufs[b][r, sl] + dbufs[b][r, sl]

            pltpu.async_copy(dbufs[b], out_slice(j), sems_w[b])

    @pl.loop(0, G_CPW_PAD, step=2)
    def _(j):
        step(j, 0)
        step(j + 1, 1)

    pltpu.make_async_copy(dbufs[0], out_slice(0), sems_w[0]).wait()


def _edge_body(g_ref, at_ref, c_ref, b1_ref, w2_ref, b2_ref, e_ref):
    ac = lax.dot_general(at_ref[...].astype(jnp.bfloat16), c_ref[...],
                         (((0,), (0,)), ((), ())),
                         preferred_element_type=jnp.float32)
    h = g_ref[...].astype(jnp.float32) + ac + b1_ref[...]
    h = jnp.maximum(h, 0.0).astype(jnp.bfloat16)
    e_ref[...] = (jnp.dot(h, w2_ref[...],
                          preferred_element_type=jnp.float32) + b2_ref[...])


def _edge_mlp(g, attr_t, c, b1, w2, b2):
    e_tile = 6400
    grid = (N_EDGES // e_tile,)
    g_spec = pl.BlockSpec((e_tile, D_FEAT), lambda t: (t, 0))
    a_spec = pl.BlockSpec((D_EDGE, e_tile), lambda t: (0, t))
    c_spec = pl.BlockSpec((D_EDGE, D_FEAT), lambda t: (0, 0))
    w_spec = pl.BlockSpec((D_FEAT, D_FEAT), lambda t: (0, 0))
    b_spec = pl.BlockSpec((1, D_FEAT), lambda t: (0, 0))
    out = jax.ShapeDtypeStruct((N_EDGES, D_FEAT), jnp.float32)
    return pl.pallas_call(
        _edge_body,
        grid=grid,
        in_specs=[g_spec, a_spec, c_spec, b_spec, w_spec, b_spec],
        out_specs=g_spec,
        out_shape=out,
    )(g, attr_t, c, b1, w2, b2)


@functools.partial(
    pl.kernel,
    out_type=jax.ShapeDtypeStruct((NC, N_NODES, D_FEAT), jnp.float32),
    mesh=_VMESH,
    scratch_types=[
        pltpu.VMEM_SHARED((N_NODES, D_FEAT), jnp.float32),
        pltpu.VMEM((ROW_CHUNK, D_FEAT), jnp.float32),
        pltpu.VMEM((IDX_RING, CHUNK), jnp.int32),
        [pltpu.VMEM((CHUNK, D_FEAT), jnp.float32)] * 2,
        [pltpu.SemaphoreType.DMA] * 2,
    ],
)
def _sc_segsum(e_hbm, dst2d_hbm, out_hbm, agg_sh, zbuf, ibuf, ebufs, sems):
    cid = lax.axis_index("c")
    sid = lax.axis_index("s")
    wid = sid * NC + cid
    crow0 = pl.multiple_of(wid * SCAT_CPS, 8)

    def valid(j):
        return jnp.logical_and(j < SCAT_CPS, wid * SCAT_CPS + j < N_CHUNKS)

    def issue_load(j, b):
        base = pl.multiple_of((wid * SCAT_CPS + j) * CHUNK, 8)
        pltpu.async_copy(e_hbm.at[pl.ds(base, CHUNK)], ebufs[b], sems[b])

    def drain_load(j, b):
        base = pl.multiple_of((wid * SCAT_CPS + j) * CHUNK, 8)
        pltpu.make_async_copy(e_hbm.at[pl.ds(base, CHUNK)], ebufs[b],
                              sems[b]).wait()

    issue_load(0, 0)

    @pl.loop(0, ROW_CHUNK)
    def _(r):
        @pl.loop(0, D_FEAT, step=16)
        def _(cc):
            zbuf[r, pl.ds(cc, 16)] = jnp.zeros((16,), jnp.float32)

    @pl.loop(0, ROW_CPS)
    def _(k):
        rchunk = sid + NS * k

        @pl.when(rchunk < N_ROW_CHUNKS)
        def _():
            rbase = pl.multiple_of(rchunk * ROW_CHUNK, 8)
            pltpu.sync_copy(zbuf, agg_sh.at[pl.ds(rbase, ROW_CHUNK)])

    plsc.subcore_barrier()

    def step(j, b):
        @pl.when(valid(j))
        def _():
            @pl.when(lax.rem(j, IDX_RING) == 0)
            def _():
                pltpu.sync_copy(
                    dst2d_hbm.at[pl.ds(pl.multiple_of(crow0 + j, 8),
                                       IDX_RING)], ibuf)

            drain_load(j, b)

            @pl.when(valid(j + 1))
            def _():
                issue_load(j + 1, 1 - b)

            pltpu.sync_copy(ebufs[b],
                            agg_sh.at[ibuf.at[lax.rem(j, IDX_RING)]],
                            add=True)

    @pl.loop(0, SCAT_CPS, step=2)
    def _(j):
        step(j, 0)
        step(j + 1, 1)

    plsc.subcore_barrier()

    @pl.loop(0, ROW_CPS)
    def _(k):
        rchunk = sid + NS * k

        @pl.when(rchunk < N_ROW_CHUNKS)
        def _():
            rbase = pl.multiple_of(rchunk * ROW_CHUNK, 8)
            sl = pl.ds(rbase, ROW_CHUNK)
            pltpu.sync_copy(agg_sh.at[sl], out_hbm.at[cid].at[sl])


def _node_body(x_ref, awp_ref, amp_ref, wn1_ref, bn1_ref, wn2_ref, bn2_ref,
               out_ref):
    aggw = awp_ref[0] + awp_ref[1]
    aggm = amp_ref[0] + amp_ref[1]
    hn = jnp.concatenate([x_ref[...], aggw, aggm],
                         axis=1).astype(jnp.bfloat16)
    h = jnp.dot(hn, wn1_ref[...], preferred_element_type=jnp.float32)
    h = jnp.maximum(h + bn1_ref[...], 0.0).astype(jnp.bfloat16)
    out_ref[...] = (jnp.dot(h, wn2_ref[...],
                            preferred_element_type=jnp.float32) + bn2_ref[...])


def _node_mlp(x_m, awp, amp, wn1, bn1, wn2, bn2):
    n_tile = 1000
    grid = (N_NODES // n_tile,)
    node_spec = pl.BlockSpec((n_tile, D_FEAT), lambda t: (t, 0))
    p_spec = pl.BlockSpec((NC, n_tile, D_FEAT), lambda t: (0, t, 0))
    wn1_spec = pl.BlockSpec((3 * D_FEAT, D_FEAT), lambda t: (0, 0))
    w_spec = pl.BlockSpec((D_FEAT, D_FEAT), lambda t: (0, 0))
    b_spec = pl.BlockSpec((1, D_FEAT), lambda t: (0, 0))
    out = jax.ShapeDtypeStruct((N_NODES, D_FEAT), jnp.float32)
    return pl.pallas_call(
        _node_body,
        grid=grid,
        in_specs=[node_spec, p_spec, p_spec,
                  wn1_spec, b_spec, w_spec, b_spec],
        out_specs=node_spec,
        out_shape=out,
    )(x_m, awp, amp, wn1, bn1, wn2, bn2)


def kernel(x_m, x_w, edge_w, edge_m, edge_attrw, edge_attrm,
           W1w, b1w, W2w, b2w, W1m, b1m, W2m, b2m,
           Wn1, bn1, Wn2, bn2):
    srcw = edge_w[0].astype(jnp.int32)
    dstw = edge_w[1].astype(jnp.int32)
    srcm = edge_m[0].astype(jnp.int32)
    dstm = edge_m[1].astype(jnp.int32)

    wcat_w = jnp.concatenate([W1w[:D_FEAT], W1w[D_FEAT:2 * D_FEAT]], axis=1)
    wcat_m = jnp.concatenate([W1m[:D_FEAT], W1m[D_FEAT:2 * D_FEAT]], axis=1)
    cw = W1w[2 * D_FEAT:]
    cm = W1m[2 * D_FEAT:]

    x_cat = jnp.concatenate([x_w, x_m], axis=0)
    w_stack = jnp.stack([wcat_w, wcat_m], axis=0).astype(jnp.bfloat16)
    s_cat, d_cat = _premix(x_cat, w_stack)

    pad = IDX_PAD - N_EDGES
    isrc_w = jnp.pad(srcw, (0, pad))
    idst_w = jnp.pad(dstw, (0, pad))
    isrc_m = jnp.pad(srcm + N_NODES, (0, pad))
    idst_m = jnp.pad(dstm + N_NODES, (0, pad))

    spad = SCAT_PAD_CHUNKS * CHUNK - N_EDGES
    dstw2d = jnp.pad(dstw, (0, spad)).reshape(SCAT_PAD_CHUNKS, CHUNK)
    dstm2d = jnp.pad(dstm, (0, spad)).reshape(SCAT_PAD_CHUNKS, CHUNK)

    bf = jnp.bfloat16
    g_w = _sc_gather(s_cat, d_cat, isrc_w, idst_w)
    ew = _edge_mlp(g_w, edge_attrw.T, cw.astype(bf), b1w.reshape(1, -1),
                   W2w.astype(bf), b2w.reshape(1, -1))
    g_m = _sc_gather(s_cat, d_cat, isrc_m, idst_m)
    em = _edge_mlp(g_m, edge_attrm.T, cm.astype(bf), b1m.reshape(1, -1),
                   W2m.astype(bf), b2m.reshape(1, -1))

    awp = _sc_segsum(ew, dstw2d)
    amp = _sc_segsum(em, dstm2d)

    x = _node_mlp(x_m, awp, amp, Wn1.astype(bf),
                  bn1.reshape(1, -1), Wn2.astype(bf), bn2.reshape(1, -1))
    return (x, ew, em)


# speedup vs baseline: 1.0158x; 1.0158x over previous
import functools

import jax
import jax.numpy as jnp
from jax import lax
from jax.experimental import pallas as pl
from jax.experimental.pallas import tpu as pltpu
from jax.experimental.pallas import tpu_sc as plsc

N_NODES = 10000
N_EDGES = 320000
D_FEAT = 128
D_EDGE = 16

NC, NS = 2, 16
NW = NC * NS
CHUNK = 128
N_CHUNKS = N_EDGES // CHUNK
SCAT_CPS = 80
SCAT_PAD_CHUNKS = NW * SCAT_CPS

G_CPW = -(-N_CHUNKS // NW)
G_CPW_PAD = G_CPW + (G_CPW % 2)
IDX_SPAN = G_CPW_PAD * CHUNK
IDX_PAD = (NW - 1) * G_CPW * CHUNK + IDX_SPAN
IDX_RING = 16
ROW_CHUNK = 40
N_ROW_CHUNKS = N_NODES // ROW_CHUNK
ROW_CPS = -(-N_ROW_CHUNKS // NS)

_VMESH = plsc.VectorSubcoreMesh(
    core_axis_name="c", subcore_axis_name="s", num_cores=NC, num_subcores=NS)


def _premix_body(x_ref, w_ref, s_ref, d_ref):
    p = jnp.dot(x_ref[...].astype(jnp.bfloat16), w_ref[0],
                preferred_element_type=jnp.float32)
    s_ref[...] = p[:, :D_FEAT]
    d_ref[...] = p[:, D_FEAT:]


def _premix(x_cat, w_stack):
    n_tile = 1000
    grid = (2 * N_NODES // n_tile,)
    node_spec = pl.BlockSpec((n_tile, D_FEAT), lambda t: (t, 0))
    w_spec = pl.BlockSpec((1, D_FEAT, 2 * D_FEAT), lambda t: (t // 10, 0, 0))
    out = jax.ShapeDtypeStruct((2 * N_NODES, D_FEAT), jnp.float32)
    return pl.pallas_call(
        _premix_body,
        grid=grid,
        in_specs=[node_spec, w_spec],
        out_specs=[node_spec] * 2,
        out_shape=[out] * 2,
    )(x_cat, w_stack)


@functools.partial(
    pl.kernel,
    out_type=jax.ShapeDtypeStruct((N_EDGES, D_FEAT), jnp.float32),
    mesh=_VMESH,
    scratch_types=[
        pltpu.VMEM((IDX_SPAN,), jnp.int32),
        pltpu.VMEM((IDX_SPAN,), jnp.int32),
        [pltpu.VMEM((CHUNK, D_FEAT), jnp.float32)] * 2,
        [pltpu.VMEM((CHUNK, D_FEAT), jnp.float32)] * 2,
        [pltpu.SemaphoreType.DMA] * 2,
        [pltpu.SemaphoreType.DMA] * 2,
    ],
)
def _sc_gather(s_tab, d_tab, isrc_hbm, idst_hbm, out_hbm,
               ibuf_s, ibuf_d, sbufs, dbufs, sems_g, sems_w):
    wid = lax.axis_index("s") * NC + lax.axis_index("c")
    span0 = pl.multiple_of(wid * (G_CPW * CHUNK), 8)

    pltpu.sync_copy(isrc_hbm.at[pl.ds(span0, IDX_SPAN)], ibuf_s)
    pltpu.sync_copy(idst_hbm.at[pl.ds(span0, IDX_SPAN)], ibuf_d)

    def rvalid(j):
        return jnp.logical_and(j < G_CPW, wid * G_CPW + j < N_CHUNKS)

    def issue_gather(j, b):
        isl = pl.ds(pl.multiple_of(j * CHUNK, 8), CHUNK)
        pltpu.async_copy(s_tab.at[ibuf_s.at[isl]], sbufs[b], sems_g[b])
        pltpu.async_copy(d_tab.at[ibuf_d.at[isl]], dbufs[b], sems_g[b])

    def drain_gather(j, b):
        isl = pl.ds(pl.multiple_of(j * CHUNK, 8), CHUNK)
        pltpu.make_async_copy(s_tab.at[ibuf_s.at[isl]], sbufs[b],
                              sems_g[b]).wait()
        pltpu.make_async_copy(d_tab.at[ibuf_d.at[isl]], dbufs[b],
                              sems_g[b]).wait()

    def out_slice(j):
        base = pl.multiple_of((wid * G_CPW + j) * CHUNK, 8)
        return out_hbm.at[pl.ds(base, CHUNK)]

    issue_gather(0, 0)

    def step(j, b):
        @pl.when(rvalid(j))
        def _():
            drain_gather(j, b)

            @pl.when(jnp.logical_and(j >= 1, rvalid(j - 1)))
            def _():
                pltpu.make_async_copy(dbufs[1 - b], out_slice(j - 1),
                                      sems_w[1 - b]).wait()

            @pl.when(rvalid(j + 1))
            def _():
                issue_gather(j + 1, 1 - b)

            @pl.loop(0, CHUNK)
            def _(r):
                for cc in range(0, D_FEAT, 16):
                    sl = pl.ds(cc, 16)
                    dbufs[b][r, sl] = sbufs[b][r, sl] + dbufs[b][r, sl]

            pltpu.async_copy(dbufs[b], out_slice(j), sems_w[b])

    @pl.loop(0, G_CPW_PAD, step=2)
    def _(j):
        step(j, 0)
        step(j + 1, 1)

    pltpu.make_async_copy(dbufs[0], out_slice(0), sems_w[0]).wait()


def _edge_body(g_ref, at_ref, c_ref, b1_ref, w2_ref, b2_ref, e_ref):
    ac = lax.dot_general(at_ref[...].astype(jnp.bfloat16), c_ref[...],
                         (((0,), (0,)), ((), ())),
                         preferred_element_type=jnp.float32)
    h = g_ref[...].astype(jnp.float32) + ac + b1_ref[...]
    h = jnp.maximum(h, 0.0).astype(jnp.bfloat16)
    e_ref[...] = (jnp.dot(h, w2_ref[...],
                          preferred_element_type=jnp.float32) + b2_ref[...])


def _edge_mlp(g, attr_t, c, b1, w2, b2):
    e_tile = 3200
    grid = (N_EDGES // e_tile,)
    g_spec = pl.BlockSpec((e_tile, D_FEAT), lambda t: (t, 0))
    a_spec = pl.BlockSpec((D_EDGE, e_tile), lambda t: (0, t))
    c_spec = pl.BlockSpec((D_EDGE, D_FEAT), lambda t: (0, 0))
    w_spec = pl.BlockSpec((D_FEAT, D_FEAT), lambda t: (0, 0))
    b_spec = pl.BlockSpec((1, D_FEAT), lambda t: (0, 0))
    out = jax.ShapeDtypeStruct((N_EDGES, D_FEAT), jnp.float32)
    return pl.pallas_call(
        _edge_body,
        grid=grid,
        in_specs=[g_spec, a_spec, c_spec, b_spec, w_spec, b_spec],
        out_specs=g_spec,
        out_shape=out,
    )(g, attr_t, c, b1, w2, b2)


@functools.partial(
    pl.kernel,
    out_type=jax.ShapeDtypeStruct((NC, N_NODES, D_FEAT), jnp.float32),
    mesh=_VMESH,
    scratch_types=[
        pltpu.VMEM_SHARED((N_NODES, D_FEAT), jnp.float32),
        pltpu.VMEM((ROW_CHUNK, D_FEAT), jnp.float32),
        pltpu.VMEM((IDX_RING, CHUNK), jnp.int32),
        [pltpu.VMEM((CHUNK, D_FEAT), jnp.float32)] * 2,
        [pltpu.SemaphoreType.DMA] * 2,
    ],
)
def _sc_segsum(e_hbm, dst2d_hbm, out_hbm, agg_sh, zbuf, ibuf, ebufs, sems):
    cid = lax.axis_index("c")
    sid = lax.axis_index("s")
    wid = sid * NC + cid
    crow0 = pl.multiple_of(wid * SCAT_CPS, 8)

    def valid(j):
        return jnp.logical_and(j < SCAT_CPS, wid * SCAT_CPS + j < N_CHUNKS)

    def issue_load(j, b):
        base = pl.multiple_of((wid * SCAT_CPS + j) * CHUNK, 8)
        pltpu.async_copy(e_hbm.at[pl.ds(base, CHUNK)], ebufs[b], sems[b])

    def drain_load(j, b):
        base = pl.multiple_of((wid * SCAT_CPS + j) * CHUNK, 8)
        pltpu.make_async_copy(e_hbm.at[pl.ds(base, CHUNK)], ebufs[b],
                              sems[b]).wait()

    issue_load(0, 0)

    @pl.loop(0, ROW_CHUNK)
    def _(r):
        @pl.loop(0, D_FEAT, step=16)
        def _(cc):
            zbuf[r, pl.ds(cc, 16)] = jnp.zeros((16,), jnp.float32)

    @pl.loop(0, ROW_CPS)
    def _(k):
        rchunk = sid + NS * k

        @pl.when(rchunk < N_ROW_CHUNKS)
        def _():
            rbase = pl.multiple_of(rchunk * ROW_CHUNK, 8)
            pltpu.sync_copy(zbuf, agg_sh.at[pl.ds(rbase, ROW_CHUNK)])

    plsc.subcore_barrier()

    def step(j, b):
        @pl.when(valid(j))
        def _():
            @pl.when(lax.rem(j, IDX_RING) == 0)
            def _():
                pltpu.sync_copy(
                    dst2d_hbm.at[pl.ds(pl.multiple_of(crow0 + j, 8),
                                       IDX_RING)], ibuf)

            drain_load(j, b)

            @pl.when(valid(j + 1))
            def _():
                issue_load(j + 1, 1 - b)

            pltpu.sync_copy(ebufs[b],
                            agg_sh.at[ibuf.at[lax.rem(j, IDX_RING)]],
                            add=True)

    @pl.loop(0, SCAT_CPS, step=2)
    def _(j):
        step(j, 0)
        step(j + 1, 1)

    plsc.subcore_barrier()

    @pl.loop(0, ROW_CPS)
    def _(k):
        rchunk = sid + NS * k

        @pl.when(rchunk < N_ROW_CHUNKS)
        def _():
            rbase = pl.multiple_of(rchunk * ROW_CHUNK, 8)
            sl = pl.ds(rbase, ROW_CHUNK)
            pltpu.sync_copy(agg_sh.at[sl], out_hbm.at[cid].at[sl])


def _node_body(x_ref, awp_ref, amp_ref, wn1_ref, bn1_ref, wn2_ref, bn2_ref,
               out_ref):
    aggw = awp_ref[0] + awp_ref[1]
    aggm = amp_ref[0] + amp_ref[1]
    hn = jnp.concatenate([x_ref[...], aggw, aggm],
                         axis=1).astype(jnp.bfloat16)
    h = jnp.dot(hn, wn1_ref[...], preferred_element_type=jnp.float32)
    h = jnp.maximum(h + bn1_ref[...], 0.0).astype(jnp.bfloat16)
    out_ref[...] = (jnp.dot(h, wn2_ref[...],
                            preferred_element_type=jnp.float32) + bn2_ref[...])


def _node_mlp(x_m, awp, amp, wn1, bn1, wn2, bn2):
    n_tile = 1000
    grid = (N_NODES // n_tile,)
    node_spec = pl.BlockSpec((n_tile, D_FEAT), lambda t: (t, 0))
    p_spec = pl.BlockSpec((NC, n_tile, D_FEAT), lambda t: (0, t, 0))
    wn1_spec = pl.BlockSpec((3 * D_FEAT, D_FEAT), lambda t: (0, 0))
    w_spec = pl.BlockSpec((D_FEAT, D_FEAT), lambda t: (0, 0))
    b_spec = pl.BlockSpec((1, D_FEAT), lambda t: (0, 0))
    out = jax.ShapeDtypeStruct((N_NODES, D_FEAT), jnp.float32)
    return pl.pallas_call(
        _node_body,
        grid=grid,
        in_specs=[node_spec, p_spec, p_spec,
                  wn1_spec, b_spec, w_spec, b_spec],
        out_specs=node_spec,
        out_shape=out,
    )(x_m, awp, amp, wn1, bn1, wn2, bn2)


def kernel(x_m, x_w, edge_w, edge_m, edge_attrw, edge_attrm,
           W1w, b1w, W2w, b2w, W1m, b1m, W2m, b2m,
           Wn1, bn1, Wn2, bn2):
    srcw = edge_w[0].astype(jnp.int32)
    dstw = edge_w[1].astype(jnp.int32)
    srcm = edge_m[0].astype(jnp.int32)
    dstm = edge_m[1].astype(jnp.int32)

    wcat_w = jnp.concatenate([W1w[:D_FEAT], W1w[D_FEAT:2 * D_FEAT]], axis=1)
    wcat_m = jnp.concatenate([W1m[:D_FEAT], W1m[D_FEAT:2 * D_FEAT]], axis=1)
    cw = W1w[2 * D_FEAT:]
    cm = W1m[2 * D_FEAT:]

    x_cat = jnp.concatenate([x_w, x_m], axis=0)
    w_stack = jnp.stack([wcat_w, wcat_m], axis=0).astype(jnp.bfloat16)
    s_cat, d_cat = _premix(x_cat, w_stack)

    pad = IDX_PAD - N_EDGES
    isrc_w = jnp.pad(srcw, (0, pad))
    idst_w = jnp.pad(dstw, (0, pad))
    isrc_m = jnp.pad(srcm + N_NODES, (0, pad))
    idst_m = jnp.pad(dstm + N_NODES, (0, pad))

    spad = SCAT_PAD_CHUNKS * CHUNK - N_EDGES
    dstw2d = jnp.pad(dstw, (0, spad)).reshape(SCAT_PAD_CHUNKS, CHUNK)
    dstm2d = jnp.pad(dstm, (0, spad)).reshape(SCAT_PAD_CHUNKS, CHUNK)

    bf = jnp.bfloat16
    g_w = _sc_gather(s_cat, d_cat, isrc_w, idst_w)
    ew = _edge_mlp(g_w, edge_attrw.T, cw.astype(bf), b1w.reshape(1, -1),
                   W2w.astype(bf), b2w.reshape(1, -1))
    g_m = _sc_gather(s_cat, d_cat, isrc_m, idst_m)
    em = _edge_mlp(g_m, edge_attrm.T, cm.astype(bf), b1m.reshape(1, -1),
                   W2m.astype(bf), b2m.reshape(1, -1))

    awp = _sc_segsum(ew, dstw2d)
    amp = _sc_segsum(em, dstm2d)

    x = _node_mlp(x_m, awp, amp, Wn1.astype(bf),
                  bn1.reshape(1, -1), Wn2.astype(bf), bn2.reshape(1, -1))
    return (x, ew, em)
